# baseline (speedup 1.0000x reference)
import functools

import jax
import jax.numpy as jnp
from jax import lax
from jax.experimental import pallas as pl
from jax.experimental.pallas import tpu as pltpu
from jax.experimental.pallas import tpu_sc as plsc


def _fuse_body(table_ref, w_ref, b_ref, out_ref):
    out_ref[...] = lax.dot_general(
        table_ref[...], w_ref[...],
        dimension_numbers=(((1,), (1,)), ((), ())),
        preferred_element_type=jnp.float32,
    ) + b_ref[...]


@functools.cache
def _build_gather(B, D):
    info = plsc.get_sparse_core_info()
    num_cores = info.num_cores
    NW = info.num_cores * info.num_subcores
    b_per_w = B // NW
    CHUNK = 128
    n_chunks = b_per_w // CHUNK
    assert B == NW * n_chunks * CHUNK
    mesh = plsc.VectorSubcoreMesh(core_axis_name="c", subcore_axis_name="s")

    @functools.partial(
        pl.kernel, mesh=mesh,
        out_type=jax.ShapeDtypeStruct((B, D), jnp.float32),
        scratch_types=[
            pltpu.VMEM((b_per_w,), jnp.int32),
            pltpu.VMEM((b_per_w, D), jnp.float32),
            pltpu.VMEM_SHARED((16, b_per_w // 2, D), jnp.float32),
            pltpu.SemaphoreType.DMA,
            pltpu.SemaphoreType.DMA,
            pltpu.SemaphoreType.DMA,
        ],
    )
    def gather(fused_hbm, idx_hbm, out_hbm, idx_v, rows_v, sp, gsem, s1, s2):
        wid = lax.axis_index("s") * num_cores + lax.axis_index("c")
        sid = lax.axis_index("s")
        half = b_per_w // 2
        base = wid * b_per_w
        pltpu.sync_copy(idx_hbm.at[pl.ds(base, b_per_w)], idx_v)
        copies = [
            pltpu.async_copy(fused_hbm.at[idx_v.at[pl.ds(j * CHUNK, CHUNK)]],
                             rows_v.at[pl.ds(j * CHUNK, CHUNK)], gsem)
            for j in range(n_chunks)
        ]
        for c in copies:
            c.wait()
        w1 = pltpu.async_copy(rows_v.at[pl.ds(0, half)],
                              out_hbm.at[pl.ds(base, half)], s1)
        pltpu.sync_copy(rows_v.at[pl.ds(half, half)], sp.at[sid])
        w2 = pltpu.async_copy(sp.at[sid],
                              out_hbm.at[pl.ds(base + half, half)], s2)
        w1.wait()
        w2.wait()

    return gather


def kernel(t, table, W, b):
    B = t.shape[0]
    V, D = table.shape
    fused = pl.pallas_call(
        _fuse_body,
        out_shape=jax.ShapeDtypeStruct((V, D), jnp.float32),
    )(table, W, b.reshape(1, D))
    return _build_gather(B, D)(fused, t)


# speedup vs baseline: 1.0433x; 1.0433x over previous
import functools

import jax
import jax.numpy as jnp
from jax import lax
from jax.experimental import pallas as pl
from jax.experimental.pallas import tpu as pltpu
from jax.experimental.pallas import tpu_sc as plsc


def _fuse_body(table_ref, w_ref, b_ref, out_ref):
    out_ref[...] = lax.dot_general(
        table_ref[...], w_ref[...],
        dimension_numbers=(((1,), (1,)), ((), ())),
        preferred_element_type=jnp.float32,
    ) + b_ref[...]


@functools.cache
def _build_gather(B, D):
    info = plsc.get_sparse_core_info()
    num_cores = info.num_cores
    NW = info.num_cores * info.num_subcores
    b_per_w = B // NW
    CHUNK = 128
    n_chunks = b_per_w // CHUNK
    assert B == NW * n_chunks * CHUNK
    mesh = plsc.VectorSubcoreMesh(core_axis_name="c", subcore_axis_name="s")

    @functools.partial(
        pl.kernel, mesh=mesh,
        out_type=jax.ShapeDtypeStruct((B, D), jnp.float32),
        scratch_types=[
            pltpu.VMEM((b_per_w,), jnp.int32),
            pltpu.VMEM((b_per_w, D), jnp.float32),
            pltpu.SemaphoreType.DMA,
        ],
    )
    def gather(fused_hbm, idx_hbm, out_hbm, idx_v, rows_v, sem):
        wid = lax.axis_index("s") * num_cores + lax.axis_index("c")
        base = wid * b_per_w
        pltpu.sync_copy(idx_hbm.at[pl.ds(base, b_per_w)], idx_v)
        copies = [
            pltpu.async_copy(fused_hbm.at[idx_v.at[pl.ds(j * CHUNK, CHUNK)]],
                             rows_v.at[pl.ds(j * CHUNK, CHUNK)], sem)
            for j in range(n_chunks)
        ]
        for c in copies:
            c.wait()
        pltpu.sync_copy(rows_v, out_hbm.at[pl.ds(base, b_per_w)])

    return gather


def kernel(t, table, W, b):
    B = t.shape[0]
    V, D = table.shape
    fused = pl.pallas_call(
        _fuse_body,
        out_shape=jax.ShapeDtypeStruct((V, D), jnp.float32),
    )(table, W, b.reshape(1, D))
    return _build_gather(B, D)(fused, t)
